# baseline (speedup 1.0000x reference)
import functools

import jax
import jax.numpy as jnp
from jax import lax
from jax.experimental import pallas as pl
from jax.experimental.pallas import tpu as pltpu
from jax.experimental.pallas import tpu_sc as plsc

_f32 = jnp.float32

_N = 10000
_E = 320000
_DIM = 128

_NC = 2
_NS = 16
_NW = _NC * _NS

_CE = 32
_EPW = _E // _NW
_NPAIR = 156
_CREM = 16
_NPAD = 10112
_ROWS_PER_TILE = _NPAD // _NS
_INV_SQRT_D = 0.08838834764831845
_SL = _INV_SQRT_D * 1.4426950408889634
_MAGIC = 12582912.0
_MAGIC_I = 1262485504
_P2 = (1.00000008, 0.69312103, 0.24022107, 0.05592204, 0.00967604)


def _tables_body(x_ref, wt12_ref, dwt_ref, swt_ref, gb_ref, db_ref, sb_ref,
                 ta_ref, tbd_ref, sl_ref):
    x = x_ref[...]
    ta_ref[...] = jnp.dot(x, wt12_ref[:_DIM], preferred_element_type=_f32)
    tbd_ref[:, :_DIM] = (
        jnp.dot(x, wt12_ref[_DIM:], preferred_element_type=_f32) + gb_ref[...])
    tbd_ref[:, _DIM:] = (
        jnp.dot(x, dwt_ref[...], preferred_element_type=_f32) + db_ref[...])
    sl_ref[...] = jnp.dot(x, swt_ref[...], preferred_element_type=_f32) + sb_ref[...]


def _ec_body(ea_ref, wt3_ref, ec_ref):
    ec_ref[...] = jnp.dot(ea_ref[...], wt3_ref[...], preferred_element_type=_f32)


def _edge_body(src_hbm, dst_hbm, ta_hbm, tbd_hbm, ec_hbm,
               m_hbm, agg_hbm, stats_hbm,
               sa0_v, da0_v, sa1_v, da1_v, sb0_v, db0_v, sb1_v, db1_v,
               se_v, de_v,
               aa_v, ab_v, bda_v, bdb_v, eca_v, ecb_v,
               m_v, cta_v, ctb_v,
               agg_sh, sem_i, sem_ga, sem_gb, sem_m, sem_cta, sem_ctb):
    c = lax.axis_index("c")
    s = lax.axis_index("s")
    base = c * (_E // _NC) + s * _EPW
    row0 = s * _ROWS_PER_TILE

    def _zct(i, _):
        r = i // 8
        sl = pl.ds((i % 8) * 16, 16)
        cta_v[r, sl] = jnp.zeros((16,), _f32)
        ctb_v[r, sl] = jnp.zeros((16,), _f32)
        return 0
    lax.fori_loop(0, _CE * 8, _zct, 0)
    nz = _ROWS_PER_TILE // _CE
    rz = _ROWS_PER_TILE - nz * _CE
    for t in range(nz):
        pltpu.async_copy(cta_v, agg_sh.at[pl.ds(row0 + t * _CE, _CE)], sem_m)
    pltpu.async_copy(cta_v.at[pl.ds(0, rz)],
                     agg_sh.at[pl.ds(row0 + nz * _CE, rz)], sem_m)
    for t in range(nz):
        pltpu.make_async_copy(
            cta_v, agg_sh.at[pl.ds(row0 + t * _CE, _CE)], sem_m).wait()
    pltpu.make_async_copy(
        cta_v.at[pl.ds(0, rz)],
        agg_sh.at[pl.ds(row0 + nz * _CE, rz)], sem_m).wait()
    plsc.subcore_barrier()

    def _issue_idx(eb, si_v, di_v, nrow):
        c1 = pltpu.async_copy(src_hbm.at[pl.ds(eb, nrow)], si_v, sem_i)
        c2 = pltpu.async_copy(dst_hbm.at[pl.ds(eb, nrow)], di_v, sem_i)
        c1.wait()
        c2.wait()

    def _issue_gathers(eb, si_v, di_v, a_v, bd_v, ec_v, sem):
        pltpu.async_copy(ta_hbm.at[si_v], a_v, sem)
        pltpu.async_copy(tbd_hbm.at[di_v], bd_v, sem)
        pltpu.async_copy(ec_hbm.at[pl.ds(eb, _CE)], ec_v, sem)

    def _wait_gathers(eb, si_v, di_v, a_v, bd_v, ec_v, sem):
        pltpu.make_async_copy(ta_hbm.at[si_v], a_v, sem).wait()
        pltpu.make_async_copy(tbd_hbm.at[di_v], bd_v, sem).wait()
        pltpu.make_async_copy(ec_hbm.at[pl.ds(eb, _CE)], ec_v, sem).wait()

    def _wait_m(nrow):
        pltpu.make_async_copy(
            m_v.at[pl.ds(0, nrow)], m_hbm.at[pl.ds(base, nrow)], sem_m).wait()

    def _wait_ct(ct_v, si_v, sem):
        pltpu.make_async_copy(ct_v, agg_sh.at[si_v], sem).wait()

    def _mk_row(a_v, bd_v, ec_v, ct_v):
        def _row(r, acc):
            accs = list(acc)
            for j in range(8):
                sl = pl.ds(j * 16, 16)
                a = a_v[r, sl]
                b = bd_v[r, sl]
                d = bd_v[r, pl.ds(_DIM + j * 16, 16)]
                e = ec_v[r, sl]
                m = a + b + e
                m_v[r, sl] = m
                z = jnp.maximum(jnp.abs(m) * (-_SL), -126.0)
                zm = z + _MAGIC
                fr = z - (zm - _MAGIC)
                ni = lax.bitcast_convert_type(zm, jnp.int32) - _MAGIC_I
                e2n = lax.bitcast_convert_type(
                    lax.shift_left(ni + 127, 23), _f32)
                p = _P2[0] + fr * (_P2[1] + fr * (_P2[2] + fr * (
                    _P2[3] + fr * _P2[4])))
                y = p * e2n + 1.0
                rc = 1.4117647 - 0.47058824 * y
                rc = rc * (2.0 - y * rc)
                rc = rc * (2.0 - y * rc)
                sig = jnp.where(m >= 0.0, rc, 1.0 - rc)
                ct_v[r, sl] = sig * d
                accs[2 * j] = accs[2 * j] + m
                accs[2 * j + 1] = accs[2 * j + 1] + m * m
            return tuple(accs)
        return _row

    _row_a = _mk_row(aa_v, bda_v, eca_v, cta_v)
    _row_b = _mk_row(ab_v, bdb_v, ecb_v, ctb_v)

    _issue_idx(base, sa0_v, da0_v, _CE)
    _issue_gathers(base, sa0_v, da0_v, aa_v, bda_v, eca_v, sem_ga)
    _issue_idx(base + _CE, sb0_v, db0_v, _CE)
    _issue_gathers(base + _CE, sb0_v, db0_v, ab_v, bdb_v, ecb_v, sem_gb)
    pltpu.async_copy(cta_v, agg_sh.at[sa0_v], sem_cta, add=True)
    pltpu.async_copy(ctb_v, agg_sh.at[sb0_v], sem_ctb, add=True)
    pltpu.async_copy(m_v, agg_hbm.at[c, pl.ds(_N + 16, _CE)], sem_m)

    def _chunk(k, acc, row_fn, a_v, bd_v, ec_v, ct_v, si_v, di_v,
               pf, pf_si, pf_di):
        eb = base + k * _CE
        _wait_gathers(eb, si_v, di_v, a_v, bd_v, ec_v,
                      sem_ga if ct_v is cta_v else sem_gb)
        _wait_ct(ct_v, si_v, sem_cta if ct_v is cta_v else sem_ctb)
        _wait_m(_CE)
        acc = lax.fori_loop(0, _CE, row_fn, acc)
        pltpu.async_copy(m_v, m_hbm.at[pl.ds(eb, _CE)], sem_m)
        pltpu.async_copy(ct_v, agg_sh.at[si_v],
                         sem_cta if ct_v is cta_v else sem_ctb, add=True)
        if pf:
            _issue_idx(eb + 2 * _CE, pf_si, pf_di, _CE)
            _issue_gathers(eb + 2 * _CE, pf_si, pf_di, a_v, bd_v, ec_v,
                           sem_ga if ct_v is cta_v else sem_gb)
        return acc

    def _quad(i, acc, pf_tail):
        k0 = 4 * i
        acc = _chunk(k0, acc, _row_a, aa_v, bda_v, eca_v, cta_v,
                     sa0_v, da0_v, True, sa1_v, da1_v)
        acc = _chunk(k0 + 1, acc, _row_b, ab_v, bdb_v, ecb_v, ctb_v,
                     sb0_v, db0_v, True, sb1_v, db1_v)
        acc = _chunk(k0 + 2, acc, _row_a, aa_v, bda_v, eca_v, cta_v,
                     sa1_v, da1_v, pf_tail, sa0_v, da0_v)
        acc = _chunk(k0 + 3, acc, _row_b, ab_v, bdb_v, ecb_v, ctb_v,
                     sb1_v, db1_v, pf_tail, sb0_v, db0_v)
        return acc

    zero16 = jnp.zeros((16,), _f32)
    nquad = _NPAIR // 2
    acc = lax.fori_loop(0, nquad - 1,
                        lambda i, a: _quad(i, a, True), (zero16,) * 16)
    acc = _quad(nquad - 1, acc, False)

    eb_r = base + 2 * _NPAIR * _CE
    _issue_idx(eb_r, se_v, de_v, _CREM)
    pltpu.async_copy(ta_hbm.at[se_v], aa_v.at[pl.ds(0, _CREM)], sem_ga)
    pltpu.async_copy(tbd_hbm.at[de_v], bda_v.at[pl.ds(0, _CREM)], sem_ga)
    pltpu.async_copy(ec_hbm.at[pl.ds(eb_r, _CREM)], eca_v.at[pl.ds(0, _CREM)],
                     sem_ga)
    _wait_ct(cta_v, sa1_v, sem_cta)
    _wait_m(_CE)
    pltpu.make_async_copy(ta_hbm.at[se_v], aa_v.at[pl.ds(0, _CREM)],
                          sem_ga).wait()
    pltpu.make_async_copy(tbd_hbm.at[de_v], bda_v.at[pl.ds(0, _CREM)],
                          sem_ga).wait()
    pltpu.make_async_copy(ec_hbm.at[pl.ds(eb_r, _CREM)],
                          eca_v.at[pl.ds(0, _CREM)], sem_ga).wait()
    acc = lax.fori_loop(0, _CREM, _row_a, acc)
    pltpu.async_copy(m_v.at[pl.ds(0, _CREM)], m_hbm.at[pl.ds(eb_r, _CREM)],
                     sem_m)
    pltpu.async_copy(cta_v.at[pl.ds(0, _CREM)], agg_sh.at[se_v], sem_cta,
                     add=True)
    _wait_m(_CREM)
    _wait_ct(ctb_v, sb1_v, sem_ctb)
    pltpu.make_async_copy(cta_v.at[pl.ds(0, _CREM)], agg_sh.at[se_v],
                          sem_cta).wait()

    def _zst(i, _):
        cta_v[i // 8, pl.ds((i % 8) * 16, 16)] = jnp.zeros((16,), _f32)
        return 0
    lax.fori_loop(0, _CREM * 8, _zst, 0)
    for j in range(8):
        cta_v[0, pl.ds(j * 16, 16)] = acc[2 * j]
        cta_v[1, pl.ds(j * 16, 16)] = acc[2 * j + 1]
    wid = s * _NC + c
    pltpu.sync_copy(cta_v.at[pl.ds(0, _CREM)], stats_hbm.at[wid])

    plsc.subcore_barrier()
    pltpu.sync_copy(agg_sh.at[pl.ds(row0, _ROWS_PER_TILE)],
                    agg_hbm.at[c, pl.ds(row0, _ROWS_PER_TILE)])


def _nodes_body(sl_ref, a0_ref, a1_ref, g_ref, b_ref, o_ref):
    h = sl_ref[...] + a0_ref[...] + a1_ref[...]
    mean = jnp.mean(h, axis=0, keepdims=True)
    var = jnp.mean((h - mean) * (h - mean), axis=0, keepdims=True)
    o_ref[...] = jnp.maximum(
        (h - mean) * lax.rsqrt(var + 1e-5) * g_ref[...] + b_ref[...], 0.0)


def _edges_bn_body(m_ref, ssum_ref, ssq_ref, g_ref, b_ref, o_ref):
    mean = jnp.sum(ssum_ref[...], axis=0, keepdims=True) * (1.0 / _E)
    msq = jnp.sum(ssq_ref[...], axis=0, keepdims=True) * (1.0 / _E)
    var = msq - mean * mean
    rstd = lax.rsqrt(var + 1e-5)
    o_ref[...] = jnp.maximum(
        (m_ref[...] - mean) * rstd * g_ref[...] + b_ref[...], 0.0)


def kernel(x, edge_index, edge_attr, gate_W, gate_b, src_W, src_b, dst_W,
           dst_b, node_gamma, node_beta, edge_gamma, edge_beta):
    src_idx = edge_index[0]
    dst_idx = edge_index[1]
    wt = gate_W.T
    wt12 = wt[:2 * _DIM]
    wt3 = wt[2 * _DIM:]
    gb2 = gate_b.reshape(1, _DIM)
    db2 = dst_b.reshape(1, _DIM)
    sb2 = src_b.reshape(1, _DIM)

    ta, tbd, src_lin = pl.pallas_call(
        _tables_body,
        out_shape=[
            jax.ShapeDtypeStruct((_N, _DIM), _f32),
            jax.ShapeDtypeStruct((_N, 2 * _DIM), _f32),
            jax.ShapeDtypeStruct((_N, _DIM), _f32),
        ],
    )(x, wt12, dst_W.T, src_W.T, gb2, db2, sb2)

    be = 4000
    ec = pl.pallas_call(
        _ec_body,
        grid=(_E // be,),
        in_specs=[
            pl.BlockSpec((be, _DIM), lambda i: (i, 0)),
            pl.BlockSpec((_DIM, _DIM), lambda i: (0, 0)),
        ],
        out_specs=pl.BlockSpec((be, _DIM), lambda i: (i, 0)),
        out_shape=jax.ShapeDtypeStruct((_E, _DIM), _f32),
    )(edge_attr, wt3)

    mesh = plsc.VectorSubcoreMesh(core_axis_name="c", subcore_axis_name="s")
    m_arr, agg2, stats = pl.kernel(
        _edge_body,
        out_type=[
            jax.ShapeDtypeStruct((_E, _DIM), _f32),
            jax.ShapeDtypeStruct((_NC, _NPAD, _DIM), _f32),
            jax.ShapeDtypeStruct((_NW, _CREM, _DIM), _f32),
        ],
        mesh=mesh,
        scratch_types=(
            [pltpu.VMEM((_CE,), jnp.int32)] * 8
            + [pltpu.VMEM((_CREM,), jnp.int32)] * 2
            + [
                pltpu.VMEM((_CE, _DIM), _f32),
                pltpu.VMEM((_CE, _DIM), _f32),
                pltpu.VMEM((_CE, 2 * _DIM), _f32),
                pltpu.VMEM((_CE, 2 * _DIM), _f32),
                pltpu.VMEM((_CE, _DIM), _f32),
                pltpu.VMEM((_CE, _DIM), _f32),
                pltpu.VMEM((_CE, _DIM), _f32),
                pltpu.VMEM((_CE, _DIM), _f32),
                pltpu.VMEM((_CE, _DIM), _f32),
                pltpu.VMEM_SHARED((_NPAD, _DIM), _f32),
            ]
            + [pltpu.SemaphoreType.DMA] * 6
        ),
    )(src_idx, dst_idx, ta, tbd, ec)

    out_nodes = pl.pallas_call(
        _nodes_body,
        out_shape=jax.ShapeDtypeStruct((_N, _DIM), _f32),
    )(src_lin, agg2[0, :_N], agg2[1, :_N], node_gamma.reshape(1, _DIM),
      node_beta.reshape(1, _DIM))

    bm = 4000
    out_edges = pl.pallas_call(
        _edges_bn_body,
        grid=(_E // bm,),
        in_specs=[
            pl.BlockSpec((bm, _DIM), lambda i: (i, 0)),
            pl.BlockSpec((_NW, _DIM), lambda i: (0, 0)),
            pl.BlockSpec((_NW, _DIM), lambda i: (0, 0)),
            pl.BlockSpec((1, _DIM), lambda i: (0, 0)),
            pl.BlockSpec((1, _DIM), lambda i: (0, 0)),
        ],
        out_specs=pl.BlockSpec((bm, _DIM), lambda i: (i, 0)),
        out_shape=jax.ShapeDtypeStruct((_E, _DIM), _f32),
    )(m_arr, stats[:, 0, :], stats[:, 1, :], edge_gamma.reshape(1, _DIM),
      edge_beta.reshape(1, _DIM))

    return (out_nodes, out_edges)


# speedup vs baseline: 1.0001x; 1.0001x over previous
import functools

import jax
import jax.numpy as jnp
from jax import lax
from jax.experimental import pallas as pl
from jax.experimental.pallas import tpu as pltpu
from jax.experimental.pallas import tpu_sc as plsc

_f32 = jnp.float32

_N = 10000
_E = 320000
_DIM = 128

_NC = 2
_NS = 16
_NW = _NC * _NS

_CE = 32
_EPW = _E // _NW
_NPAIR = 156
_CREM = 16
_NPAD = 10112
_ROWS_PER_TILE = _NPAD // _NS
_INV_SQRT_D = 0.08838834764831845
_SL = _INV_SQRT_D * 1.4426950408889634
_MAGIC = 12582912.0
_MAGIC_I = 1262485504
_P2 = (1.00000008, 0.69312103, 0.24022107, 0.05592204, 0.00967604)


def _tables_body(x_ref, wt12_ref, dwt_ref, swt_ref, gb_ref, db_ref, sb_ref,
                 ta_ref, tbd_ref, sl_ref):
    x = x_ref[...]
    ta_ref[...] = jnp.dot(x, wt12_ref[:_DIM], preferred_element_type=_f32)
    tbd_ref[:, :_DIM] = (
        jnp.dot(x, wt12_ref[_DIM:], preferred_element_type=_f32) + gb_ref[...])
    tbd_ref[:, _DIM:] = (
        jnp.dot(x, dwt_ref[...], preferred_element_type=_f32) + db_ref[...])
    sl_ref[...] = jnp.dot(x, swt_ref[...], preferred_element_type=_f32) + sb_ref[...]


def _ec_body(ea_ref, wt3_ref, ec_ref):
    ec_ref[...] = jnp.dot(ea_ref[...], wt3_ref[...], preferred_element_type=_f32)


def _edge_body(src_hbm, dst_hbm, ta_hbm, tbd_hbm, ec_hbm,
               m_hbm, agg_hbm, stats_hbm,
               sa0_v, da0_v, sa1_v, da1_v, sb0_v, db0_v, sb1_v, db1_v,
               se_v, de_v,
               aa_v, ab_v, bda_v, bdb_v, eca_v, ecb_v,
               m_v, cta_v, ctb_v,
               agg_sh, sem_i, sem_ga, sem_gb, sem_m, sem_cta, sem_ctb):
    c = lax.axis_index("c")
    s = lax.axis_index("s")
    base = c * (_E // _NC) + s * _EPW
    row0 = s * _ROWS_PER_TILE

    def _zct(i, _):
        r = i // 8
        sl = pl.ds((i % 8) * 16, 16)
        cta_v[r, sl] = jnp.zeros((16,), _f32)
        ctb_v[r, sl] = jnp.zeros((16,), _f32)
        return 0
    lax.fori_loop(0, _CE * 8, _zct, 0)
    nz = _ROWS_PER_TILE // _CE
    rz = _ROWS_PER_TILE - nz * _CE
    for t in range(nz):
        pltpu.async_copy(cta_v, agg_sh.at[pl.ds(row0 + t * _CE, _CE)], sem_m)
    pltpu.async_copy(cta_v.at[pl.ds(0, rz)],
                     agg_sh.at[pl.ds(row0 + nz * _CE, rz)], sem_m)
    for t in range(nz):
        pltpu.make_async_copy(
            cta_v, agg_sh.at[pl.ds(row0 + t * _CE, _CE)], sem_m).wait()
    pltpu.make_async_copy(
        cta_v.at[pl.ds(0, rz)],
        agg_sh.at[pl.ds(row0 + nz * _CE, rz)], sem_m).wait()
    plsc.subcore_barrier()

    def _issue_idx(eb, si_v, di_v, nrow):
        c1 = pltpu.async_copy(src_hbm.at[pl.ds(eb, nrow)], si_v, sem_i)
        c2 = pltpu.async_copy(dst_hbm.at[pl.ds(eb, nrow)], di_v, sem_i)
        c1.wait()
        c2.wait()

    def _issue_gathers(eb, si_v, di_v, a_v, bd_v, ec_v, sem):
        pltpu.async_copy(ta_hbm.at[si_v], a_v, sem)
        pltpu.async_copy(tbd_hbm.at[di_v], bd_v, sem)
        pltpu.async_copy(ec_hbm.at[pl.ds(eb, _CE)], ec_v, sem)

    def _wait_gathers(eb, si_v, di_v, a_v, bd_v, ec_v, sem):
        pltpu.make_async_copy(ta_hbm.at[si_v], a_v, sem).wait()
        pltpu.make_async_copy(tbd_hbm.at[di_v], bd_v, sem).wait()
        pltpu.make_async_copy(ec_hbm.at[pl.ds(eb, _CE)], ec_v, sem).wait()

    def _wait_m(nrow):
        pltpu.make_async_copy(
            m_v.at[pl.ds(0, nrow)], m_hbm.at[pl.ds(base, nrow)], sem_m).wait()

    def _wait_ct(ct_v, si_v, sem):
        pltpu.make_async_copy(ct_v, agg_sh.at[si_v], sem).wait()

    def _mk_row(a_v, bd_v, ec_v, ct_v):
        def _row(r, acc):
            accs = list(acc)
            for j in range(8):
                sl = pl.ds(j * 16, 16)
                a = a_v[r, sl]
                b = bd_v[r, sl]
                d = bd_v[r, pl.ds(_DIM + j * 16, 16)]
                e = ec_v[r, sl]
                m = a + b + e
                m_v[r, sl] = m
                z = jnp.maximum(jnp.abs(m) * (-_SL), -126.0)
                zm = z + _MAGIC
                fr = z - (zm - _MAGIC)
                ni = lax.bitcast_convert_type(zm, jnp.int32) - _MAGIC_I
                e2n = lax.bitcast_convert_type(
                    lax.shift_left(ni + 127, 23), _f32)
                p = _P2[0] + fr * (_P2[1] + fr * (_P2[2] + fr * (
                    _P2[3] + fr * _P2[4])))
                y = p * e2n + 1.0
                rc = 1.4117647 - 0.47058824 * y
                rc = rc * (2.0 - y * rc)
                rc = rc * (2.0 - y * rc)
                sig = jnp.where(m >= 0.0, rc, 1.0 - rc)
                ct_v[r, sl] = sig * d
                accs[2 * j] = accs[2 * j] + m
                accs[2 * j + 1] = accs[2 * j + 1] + m * m
            return tuple(accs)
        return _row

    _row_a = _mk_row(aa_v, bda_v, eca_v, cta_v)
    _row_b = _mk_row(ab_v, bdb_v, ecb_v, ctb_v)

    _issue_idx(base, sa0_v, da0_v, _CE)
    _issue_gathers(base, sa0_v, da0_v, aa_v, bda_v, eca_v, sem_ga)
    _issue_idx(base + _CE, sb0_v, db0_v, _CE)
    _issue_gathers(base + _CE, sb0_v, db0_v, ab_v, bdb_v, ecb_v, sem_gb)
    pltpu.async_copy(cta_v, agg_sh.at[sa0_v], sem_cta, add=True)
    pltpu.async_copy(ctb_v, agg_sh.at[sb0_v], sem_ctb, add=True)
    pltpu.async_copy(m_v, agg_hbm.at[c, pl.ds(_N + 16, _CE)], sem_m)

    def _chunk(k, acc, row_fn, a_v, bd_v, ec_v, ct_v, si_v, di_v,
               pf, pf_si, pf_di):
        eb = base + k * _CE
        _wait_gathers(eb, si_v, di_v, a_v, bd_v, ec_v,
                      sem_ga if ct_v is cta_v else sem_gb)
        _wait_ct(ct_v, si_v, sem_cta if ct_v is cta_v else sem_ctb)
        _wait_m(_CE)
        acc = plsc.parallel_loop(0, _CE, carry=acc)(row_fn)
        pltpu.async_copy(m_v, m_hbm.at[pl.ds(eb, _CE)], sem_m)
        pltpu.async_copy(ct_v, agg_sh.at[si_v],
                         sem_cta if ct_v is cta_v else sem_ctb, add=True)
        if pf:
            _issue_idx(eb + 2 * _CE, pf_si, pf_di, _CE)
            _issue_gathers(eb + 2 * _CE, pf_si, pf_di, a_v, bd_v, ec_v,
                           sem_ga if ct_v is cta_v else sem_gb)
        return acc

    def _quad(i, acc, pf_tail):
        k0 = 4 * i
        acc = _chunk(k0, acc, _row_a, aa_v, bda_v, eca_v, cta_v,
                     sa0_v, da0_v, True, sa1_v, da1_v)
        acc = _chunk(k0 + 1, acc, _row_b, ab_v, bdb_v, ecb_v, ctb_v,
                     sb0_v, db0_v, True, sb1_v, db1_v)
        acc = _chunk(k0 + 2, acc, _row_a, aa_v, bda_v, eca_v, cta_v,
                     sa1_v, da1_v, pf_tail, sa0_v, da0_v)
        acc = _chunk(k0 + 3, acc, _row_b, ab_v, bdb_v, ecb_v, ctb_v,
                     sb1_v, db1_v, pf_tail, sb0_v, db0_v)
        return acc

    zero16 = jnp.zeros((16,), _f32)
    nquad = _NPAIR // 2
    acc = lax.fori_loop(0, nquad - 1,
                        lambda i, a: _quad(i, a, True), (zero16,) * 16)
    acc = _quad(nquad - 1, acc, False)

    eb_r = base + 2 * _NPAIR * _CE
    _issue_idx(eb_r, se_v, de_v, _CREM)
    pltpu.async_copy(ta_hbm.at[se_v], aa_v.at[pl.ds(0, _CREM)], sem_ga)
    pltpu.async_copy(tbd_hbm.at[de_v], bda_v.at[pl.ds(0, _CREM)], sem_ga)
    pltpu.async_copy(ec_hbm.at[pl.ds(eb_r, _CREM)], eca_v.at[pl.ds(0, _CREM)],
                     sem_ga)
    _wait_ct(cta_v, sa1_v, sem_cta)
    _wait_m(_CE)
    pltpu.make_async_copy(ta_hbm.at[se_v], aa_v.at[pl.ds(0, _CREM)],
                          sem_ga).wait()
    pltpu.make_async_copy(tbd_hbm.at[de_v], bda_v.at[pl.ds(0, _CREM)],
                          sem_ga).wait()
    pltpu.make_async_copy(ec_hbm.at[pl.ds(eb_r, _CREM)],
                          eca_v.at[pl.ds(0, _CREM)], sem_ga).wait()
    acc = plsc.parallel_loop(0, _CREM, carry=acc)(_row_a)
    pltpu.async_copy(m_v.at[pl.ds(0, _CREM)], m_hbm.at[pl.ds(eb_r, _CREM)],
                     sem_m)
    pltpu.async_copy(cta_v.at[pl.ds(0, _CREM)], agg_sh.at[se_v], sem_cta,
                     add=True)
    _wait_m(_CREM)
    _wait_ct(ctb_v, sb1_v, sem_ctb)
    pltpu.make_async_copy(cta_v.at[pl.ds(0, _CREM)], agg_sh.at[se_v],
                          sem_cta).wait()

    def _zst(i, _):
        cta_v[i // 8, pl.ds((i % 8) * 16, 16)] = jnp.zeros((16,), _f32)
        return 0
    lax.fori_loop(0, _CREM * 8, _zst, 0)
    for j in range(8):
        cta_v[0, pl.ds(j * 16, 16)] = acc[2 * j]
        cta_v[1, pl.ds(j * 16, 16)] = acc[2 * j + 1]
    wid = s * _NC + c
    pltpu.sync_copy(cta_v.at[pl.ds(0, _CREM)], stats_hbm.at[wid])

    plsc.subcore_barrier()
    pltpu.sync_copy(agg_sh.at[pl.ds(row0, _ROWS_PER_TILE)],
                    agg_hbm.at[c, pl.ds(row0, _ROWS_PER_TILE)])


def _nodes_body(sl_ref, a0_ref, a1_ref, g_ref, b_ref, o_ref):
    h = sl_ref[...] + a0_ref[...] + a1_ref[...]
    mean = jnp.mean(h, axis=0, keepdims=True)
    var = jnp.mean((h - mean) * (h - mean), axis=0, keepdims=True)
    o_ref[...] = jnp.maximum(
        (h - mean) * lax.rsqrt(var + 1e-5) * g_ref[...] + b_ref[...], 0.0)


def _edges_bn_body(m_ref, ssum_ref, ssq_ref, g_ref, b_ref, o_ref):
    mean = jnp.sum(ssum_ref[...], axis=0, keepdims=True) * (1.0 / _E)
    msq = jnp.sum(ssq_ref[...], axis=0, keepdims=True) * (1.0 / _E)
    var = msq - mean * mean
    rstd = lax.rsqrt(var + 1e-5)
    o_ref[...] = jnp.maximum(
        (m_ref[...] - mean) * rstd * g_ref[...] + b_ref[...], 0.0)


def kernel(x, edge_index, edge_attr, gate_W, gate_b, src_W, src_b, dst_W,
           dst_b, node_gamma, node_beta, edge_gamma, edge_beta):
    src_idx = edge_index[0]
    dst_idx = edge_index[1]
    wt = gate_W.T
    wt12 = wt[:2 * _DIM]
    wt3 = wt[2 * _DIM:]
    gb2 = gate_b.reshape(1, _DIM)
    db2 = dst_b.reshape(1, _DIM)
    sb2 = src_b.reshape(1, _DIM)

    ta, tbd, src_lin = pl.pallas_call(
        _tables_body,
        out_shape=[
            jax.ShapeDtypeStruct((_N, _DIM), _f32),
            jax.ShapeDtypeStruct((_N, 2 * _DIM), _f32),
            jax.ShapeDtypeStruct((_N, _DIM), _f32),
        ],
    )(x, wt12, dst_W.T, src_W.T, gb2, db2, sb2)

    be = 4000
    ec = pl.pallas_call(
        _ec_body,
        grid=(_E // be,),
        in_specs=[
            pl.BlockSpec((be, _DIM), lambda i: (i, 0)),
            pl.BlockSpec((_DIM, _DIM), lambda i: (0, 0)),
        ],
        out_specs=pl.BlockSpec((be, _DIM), lambda i: (i, 0)),
        out_shape=jax.ShapeDtypeStruct((_E, _DIM), _f32),
    )(edge_attr, wt3)

    mesh = plsc.VectorSubcoreMesh(core_axis_name="c", subcore_axis_name="s")
    m_arr, agg2, stats = pl.kernel(
        _edge_body,
        out_type=[
            jax.ShapeDtypeStruct((_E, _DIM), _f32),
            jax.ShapeDtypeStruct((_NC, _NPAD, _DIM), _f32),
            jax.ShapeDtypeStruct((_NW, _CREM, _DIM), _f32),
        ],
        mesh=mesh,
        scratch_types=(
            [pltpu.VMEM((_CE,), jnp.int32)] * 8
            + [pltpu.VMEM((_CREM,), jnp.int32)] * 2
            + [
                pltpu.VMEM((_CE, _DIM), _f32),
                pltpu.VMEM((_CE, _DIM), _f32),
                pltpu.VMEM((_CE, 2 * _DIM), _f32),
                pltpu.VMEM((_CE, 2 * _DIM), _f32),
                pltpu.VMEM((_CE, _DIM), _f32),
                pltpu.VMEM((_CE, _DIM), _f32),
                pltpu.VMEM((_CE, _DIM), _f32),
                pltpu.VMEM((_CE, _DIM), _f32),
                pltpu.VMEM((_CE, _DIM), _f32),
                pltpu.VMEM_SHARED((_NPAD, _DIM), _f32),
            ]
            + [pltpu.SemaphoreType.DMA] * 6
        ),
    )(src_idx, dst_idx, ta, tbd, ec)

    out_nodes = pl.pallas_call(
        _nodes_body,
        out_shape=jax.ShapeDtypeStruct((_N, _DIM), _f32),
    )(src_lin, agg2[0, :_N], agg2[1, :_N], node_gamma.reshape(1, _DIM),
      node_beta.reshape(1, _DIM))

    bm = 4000
    out_edges = pl.pallas_call(
        _edges_bn_body,
        grid=(_E // bm,),
        in_specs=[
            pl.BlockSpec((bm, _DIM), lambda i: (i, 0)),
            pl.BlockSpec((_NW, _DIM), lambda i: (0, 0)),
            pl.BlockSpec((_NW, _DIM), lambda i: (0, 0)),
            pl.BlockSpec((1, _DIM), lambda i: (0, 0)),
            pl.BlockSpec((1, _DIM), lambda i: (0, 0)),
        ],
        out_specs=pl.BlockSpec((bm, _DIM), lambda i: (i, 0)),
        out_shape=jax.ShapeDtypeStruct((_E, _DIM), _f32),
    )(m_arr, stats[:, 0, :], stats[:, 1, :], edge_gamma.reshape(1, _DIM),
      edge_beta.reshape(1, _DIM))

    return (out_nodes, out_edges)


# speedup vs baseline: 2.5657x; 2.5656x over previous
import functools

import jax
import jax.numpy as jnp
from jax import lax
from jax.experimental import pallas as pl
from jax.experimental.pallas import tpu as pltpu
from jax.experimental.pallas import tpu_sc as plsc

_f32 = jnp.float32

_N = 10000
_E = 320000
_DIM = 128

_NC = 2
_NS = 16
_NW = _NC * _NS

_CE = 32
_EPW = _E // _NW
_NPAIR = 156
_CREM = 16
_NPAD = 10112
_ROWS_PER_TILE = _NPAD // _NS
_INV_SQRT_D = 0.08838834764831845
_SL = _INV_SQRT_D * 1.4426950408889634
_MAGIC = 12582912.0
_MAGIC_I = 1262485504
_P2 = (1.00000008, 0.69312103, 0.24022107, 0.05592204, 0.00967604)


def _tables_body(x_ref, wt12_ref, dwt_ref, swt_ref, gb_ref, db_ref, sb_ref,
                 ta_ref, tbd_ref, sl_ref):
    x = x_ref[...]
    ta_ref[...] = jnp.dot(x, wt12_ref[:_DIM], preferred_element_type=_f32)
    tbd_ref[:, :_DIM] = (
        jnp.dot(x, wt12_ref[_DIM:], preferred_element_type=_f32) + gb_ref[...])
    tbd_ref[:, _DIM:] = (
        jnp.dot(x, dwt_ref[...], preferred_element_type=_f32) + db_ref[...])
    sl_ref[...] = jnp.dot(x, swt_ref[...], preferred_element_type=_f32) + sb_ref[...]


def _ec_body(ea_ref, wt3_ref, ec_ref):
    ec_ref[...] = jnp.dot(ea_ref[...], wt3_ref[...], preferred_element_type=_f32)


def _edge_body(src_hbm, dst_hbm, ta_hbm, tbd_hbm, ec_hbm,
               m_hbm, agg_hbm, stats_hbm,
               sa0_v, da0_v, sa1_v, da1_v, sb0_v, db0_v, sb1_v, db1_v,
               se_v, de_v,
               aa_v, ab_v, bda_v, bdb_v, eca_v, ecb_v,
               m_v, cta_v, ctb_v,
               agg_sh, sem_i, sem_ga, sem_gb, sem_m, sem_cta, sem_ctb):
    c = lax.axis_index("c")
    s = lax.axis_index("s")
    base = c * (_E // _NC) + s * _EPW
    row0 = s * _ROWS_PER_TILE

    def _zct(i, _):
        r = i // 8
        sl = pl.ds((i % 8) * 16, 16)
        cta_v[r, sl] = jnp.zeros((16,), _f32)
        ctb_v[r, sl] = jnp.zeros((16,), _f32)
        return 0
    lax.fori_loop(0, _CE * 8, _zct, 0)
    nz = _ROWS_PER_TILE // _CE
    rz = _ROWS_PER_TILE - nz * _CE
    for t in range(nz):
        pltpu.async_copy(cta_v, agg_sh.at[pl.ds(row0 + t * _CE, _CE)], sem_m)
    pltpu.async_copy(cta_v.at[pl.ds(0, rz)],
                     agg_sh.at[pl.ds(row0 + nz * _CE, rz)], sem_m)
    for t in range(nz):
        pltpu.make_async_copy(
            cta_v, agg_sh.at[pl.ds(row0 + t * _CE, _CE)], sem_m).wait()
    pltpu.make_async_copy(
        cta_v.at[pl.ds(0, rz)],
        agg_sh.at[pl.ds(row0 + nz * _CE, rz)], sem_m).wait()
    plsc.subcore_barrier()

    def _issue_idx(eb, si_v, di_v, nrow):
        c1 = pltpu.async_copy(src_hbm.at[pl.ds(eb, nrow)], si_v, sem_i)
        c2 = pltpu.async_copy(dst_hbm.at[pl.ds(eb, nrow)], di_v, sem_i)
        c1.wait()
        c2.wait()

    def _issue_gathers(eb, si_v, di_v, a_v, bd_v, ec_v, sem):
        pltpu.async_copy(ta_hbm.at[si_v], a_v, sem)
        pltpu.async_copy(tbd_hbm.at[di_v], bd_v, sem)
        pltpu.async_copy(ec_hbm.at[pl.ds(eb, _CE)], ec_v, sem)

    def _wait_gathers(eb, si_v, di_v, a_v, bd_v, ec_v, sem):
        pltpu.make_async_copy(ta_hbm.at[si_v], a_v, sem).wait()
        pltpu.make_async_copy(tbd_hbm.at[di_v], bd_v, sem).wait()
        pltpu.make_async_copy(ec_hbm.at[pl.ds(eb, _CE)], ec_v, sem).wait()

    def _wait_m(nrow):
        pltpu.make_async_copy(
            m_v.at[pl.ds(0, nrow)], m_hbm.at[pl.ds(base, nrow)], sem_m).wait()

    def _wait_ct(ct_v, si_v, sem):
        pltpu.make_async_copy(ct_v, agg_sh.at[si_v], sem).wait()

    def _mk_row(a_v, bd_v, ec_v, ct_v):
        def _row(r, acc):
            accs = list(acc)
            for blk in range(2):
                js = range(4 * blk, 4 * blk + 4)
                sls = [pl.ds(j * 16, 16) for j in js]
                a = [a_v[r, sl] for sl in sls]
                b = [bd_v[r, sl] for sl in sls]
                e = [ec_v[r, sl] for sl in sls]
                m = [a[t] + b[t] for t in range(4)]
                m = [m[t] + e[t] for t in range(4)]
                for t, j in enumerate(js):
                    m_v[r, sls[t]] = m[t]
                am = [jnp.abs(m[t]) for t in range(4)]
                z = [am[t] * (-_SL) for t in range(4)]
                z = [jnp.maximum(z[t], -126.0) for t in range(4)]
                zm = [z[t] + _MAGIC for t in range(4)]
                nf = [zm[t] - _MAGIC for t in range(4)]
                fr = [z[t] - nf[t] for t in range(4)]
                ni = [lax.bitcast_convert_type(zm[t], jnp.int32) - _MAGIC_I
                      for t in range(4)]
                sh = [lax.shift_left(ni[t] + 127, 23) for t in range(4)]
                e2 = [lax.bitcast_convert_type(sh[t], _f32) for t in range(4)]
                p = [fr[t] * _P2[4] + _P2[3] for t in range(4)]
                p = [p[t] * fr[t] + _P2[2] for t in range(4)]
                p = [p[t] * fr[t] + _P2[1] for t in range(4)]
                p = [p[t] * fr[t] + _P2[0] for t in range(4)]
                y = [p[t] * e2[t] + 1.0 for t in range(4)]
                rc = [1.4117647 - 0.47058824 * y[t] for t in range(4)]
                w = [2.0 - y[t] * rc[t] for t in range(4)]
                rc = [rc[t] * w[t] for t in range(4)]
                w = [2.0 - y[t] * rc[t] for t in range(4)]
                rc = [rc[t] * w[t] for t in range(4)]
                ge = [m[t] >= 0.0 for t in range(4)]
                om = [1.0 - rc[t] for t in range(4)]
                sg = [jnp.where(ge[t], rc[t], om[t]) for t in range(4)]
                d = [bd_v[r, pl.ds(_DIM + j * 16, 16)] for j in js]
                ct = [sg[t] * d[t] for t in range(4)]
                for t, j in enumerate(js):
                    ct_v[r, sls[t]] = ct[t]
                for t, j in enumerate(js):
                    accs[2 * j] = accs[2 * j] + m[t]
                sq = [m[t] * m[t] for t in range(4)]
                for t, j in enumerate(js):
                    accs[2 * j + 1] = accs[2 * j + 1] + sq[t]
            return tuple(accs)
        return _row

    _row_a = _mk_row(aa_v, bda_v, eca_v, cta_v)
    _row_b = _mk_row(ab_v, bdb_v, ecb_v, ctb_v)

    _issue_idx(base, sa0_v, da0_v, _CE)
    _issue_gathers(base, sa0_v, da0_v, aa_v, bda_v, eca_v, sem_ga)
    _issue_idx(base + _CE, sb0_v, db0_v, _CE)
    _issue_gathers(base + _CE, sb0_v, db0_v, ab_v, bdb_v, ecb_v, sem_gb)
    pltpu.async_copy(cta_v, agg_sh.at[sa0_v], sem_cta, add=True)
    pltpu.async_copy(ctb_v, agg_sh.at[sb0_v], sem_ctb, add=True)
    pltpu.async_copy(m_v, agg_hbm.at[c, pl.ds(_N + 16, _CE)], sem_m)

    def _chunk(k, acc, row_fn, a_v, bd_v, ec_v, ct_v, si_v, di_v,
               pf, pf_si, pf_di):
        eb = base + k * _CE
        _wait_gathers(eb, si_v, di_v, a_v, bd_v, ec_v,
                      sem_ga if ct_v is cta_v else sem_gb)
        _wait_ct(ct_v, si_v, sem_cta if ct_v is cta_v else sem_ctb)
        _wait_m(_CE)
        acc = plsc.parallel_loop(0, _CE, carry=acc)(row_fn)
        pltpu.async_copy(m_v, m_hbm.at[pl.ds(eb, _CE)], sem_m)
        pltpu.async_copy(ct_v, agg_sh.at[si_v],
                         sem_cta if ct_v is cta_v else sem_ctb, add=True)
        if pf:
            _issue_idx(eb + 2 * _CE, pf_si, pf_di, _CE)
            _issue_gathers(eb + 2 * _CE, pf_si, pf_di, a_v, bd_v, ec_v,
                           sem_ga if ct_v is cta_v else sem_gb)
        return acc

    def _quad(i, acc, pf_tail):
        k0 = 4 * i
        acc = _chunk(k0, acc, _row_a, aa_v, bda_v, eca_v, cta_v,
                     sa0_v, da0_v, True, sa1_v, da1_v)
        acc = _chunk(k0 + 1, acc, _row_b, ab_v, bdb_v, ecb_v, ctb_v,
                     sb0_v, db0_v, True, sb1_v, db1_v)
        acc = _chunk(k0 + 2, acc, _row_a, aa_v, bda_v, eca_v, cta_v,
                     sa1_v, da1_v, pf_tail, sa0_v, da0_v)
        acc = _chunk(k0 + 3, acc, _row_b, ab_v, bdb_v, ecb_v, ctb_v,
                     sb1_v, db1_v, pf_tail, sb0_v, db0_v)
        return acc

    zero16 = jnp.zeros((16,), _f32)
    nquad = _NPAIR // 2
    acc = lax.fori_loop(0, nquad - 1,
                        lambda i, a: _quad(i, a, True), (zero16,) * 16)
    acc = _quad(nquad - 1, acc, False)

    eb_r = base + 2 * _NPAIR * _CE
    _issue_idx(eb_r, se_v, de_v, _CREM)
    pltpu.async_copy(ta_hbm.at[se_v], aa_v.at[pl.ds(0, _CREM)], sem_ga)
    pltpu.async_copy(tbd_hbm.at[de_v], bda_v.at[pl.ds(0, _CREM)], sem_ga)
    pltpu.async_copy(ec_hbm.at[pl.ds(eb_r, _CREM)], eca_v.at[pl.ds(0, _CREM)],
                     sem_ga)
    _wait_ct(cta_v, sa1_v, sem_cta)
    _wait_m(_CE)
    pltpu.make_async_copy(ta_hbm.at[se_v], aa_v.at[pl.ds(0, _CREM)],
                          sem_ga).wait()
    pltpu.make_async_copy(tbd_hbm.at[de_v], bda_v.at[pl.ds(0, _CREM)],
                          sem_ga).wait()
    pltpu.make_async_copy(ec_hbm.at[pl.ds(eb_r, _CREM)],
                          eca_v.at[pl.ds(0, _CREM)], sem_ga).wait()
    acc = plsc.parallel_loop(0, _CREM, carry=acc)(_row_a)
    pltpu.async_copy(m_v.at[pl.ds(0, _CREM)], m_hbm.at[pl.ds(eb_r, _CREM)],
                     sem_m)
    pltpu.async_copy(cta_v.at[pl.ds(0, _CREM)], agg_sh.at[se_v], sem_cta,
                     add=True)
    _wait_m(_CREM)
    _wait_ct(ctb_v, sb1_v, sem_ctb)
    pltpu.make_async_copy(cta_v.at[pl.ds(0, _CREM)], agg_sh.at[se_v],
                          sem_cta).wait()

    def _zst(i, _):
        cta_v[i // 8, pl.ds((i % 8) * 16, 16)] = jnp.zeros((16,), _f32)
        return 0
    lax.fori_loop(0, _CREM * 8, _zst, 0)
    for j in range(8):
        cta_v[0, pl.ds(j * 16, 16)] = acc[2 * j]
        cta_v[1, pl.ds(j * 16, 16)] = acc[2 * j + 1]
    wid = s * _NC + c
    pltpu.sync_copy(cta_v.at[pl.ds(0, _CREM)], stats_hbm.at[wid])

    plsc.subcore_barrier()
    pltpu.sync_copy(agg_sh.at[pl.ds(row0, _ROWS_PER_TILE)],
                    agg_hbm.at[c, pl.ds(row0, _ROWS_PER_TILE)])


def _nodes_body(sl_ref, a0_ref, a1_ref, g_ref, b_ref, o_ref):
    h = sl_ref[...] + a0_ref[...] + a1_ref[...]
    mean = jnp.mean(h, axis=0, keepdims=True)
    var = jnp.mean((h - mean) * (h - mean), axis=0, keepdims=True)
    o_ref[...] = jnp.maximum(
        (h - mean) * lax.rsqrt(var + 1e-5) * g_ref[...] + b_ref[...], 0.0)


def _edges_bn_body(m_ref, ssum_ref, ssq_ref, g_ref, b_ref, o_ref):
    mean = jnp.sum(ssum_ref[...], axis=0, keepdims=True) * (1.0 / _E)
    msq = jnp.sum(ssq_ref[...], axis=0, keepdims=True) * (1.0 / _E)
    var = msq - mean * mean
    rstd = lax.rsqrt(var + 1e-5)
    o_ref[...] = jnp.maximum(
        (m_ref[...] - mean) * rstd * g_ref[...] + b_ref[...], 0.0)


def kernel(x, edge_index, edge_attr, gate_W, gate_b, src_W, src_b, dst_W,
           dst_b, node_gamma, node_beta, edge_gamma, edge_beta):
    src_idx = edge_index[0]
    dst_idx = edge_index[1]
    wt = gate_W.T
    wt12 = wt[:2 * _DIM]
    wt3 = wt[2 * _DIM:]
    gb2 = gate_b.reshape(1, _DIM)
    db2 = dst_b.reshape(1, _DIM)
    sb2 = src_b.reshape(1, _DIM)

    ta, tbd, src_lin = pl.pallas_call(
        _tables_body,
        out_shape=[
            jax.ShapeDtypeStruct((_N, _DIM), _f32),
            jax.ShapeDtypeStruct((_N, 2 * _DIM), _f32),
            jax.ShapeDtypeStruct((_N, _DIM), _f32),
        ],
    )(x, wt12, dst_W.T, src_W.T, gb2, db2, sb2)

    be = 4000
    ec = pl.pallas_call(
        _ec_body,
        grid=(_E // be,),
        in_specs=[
            pl.BlockSpec((be, _DIM), lambda i: (i, 0)),
            pl.BlockSpec((_DIM, _DIM), lambda i: (0, 0)),
        ],
        out_specs=pl.BlockSpec((be, _DIM), lambda i: (i, 0)),
        out_shape=jax.ShapeDtypeStruct((_E, _DIM), _f32),
    )(edge_attr, wt3)

    mesh = plsc.VectorSubcoreMesh(core_axis_name="c", subcore_axis_name="s")
    m_arr, agg2, stats = pl.kernel(
        _edge_body,
        out_type=[
            jax.ShapeDtypeStruct((_E, _DIM), _f32),
            jax.ShapeDtypeStruct((_NC, _NPAD, _DIM), _f32),
            jax.ShapeDtypeStruct((_NW, _CREM, _DIM), _f32),
        ],
        mesh=mesh,
        scratch_types=(
            [pltpu.VMEM((_CE,), jnp.int32)] * 8
            + [pltpu.VMEM((_CREM,), jnp.int32)] * 2
            + [
                pltpu.VMEM((_CE, _DIM), _f32),
                pltpu.VMEM((_CE, _DIM), _f32),
                pltpu.VMEM((_CE, 2 * _DIM), _f32),
                pltpu.VMEM((_CE, 2 * _DIM), _f32),
                pltpu.VMEM((_CE, _DIM), _f32),
                pltpu.VMEM((_CE, _DIM), _f32),
                pltpu.VMEM((_CE, _DIM), _f32),
                pltpu.VMEM((_CE, _DIM), _f32),
                pltpu.VMEM((_CE, _DIM), _f32),
                pltpu.VMEM_SHARED((_NPAD, _DIM), _f32),
            ]
            + [pltpu.SemaphoreType.DMA] * 6
        ),
    )(src_idx, dst_idx, ta, tbd, ec)

    out_nodes = pl.pallas_call(
        _nodes_body,
        out_shape=jax.ShapeDtypeStruct((_N, _DIM), _f32),
    )(src_lin, agg2[0, :_N], agg2[1, :_N], node_gamma.reshape(1, _DIM),
      node_beta.reshape(1, _DIM))

    bm = 4000
    out_edges = pl.pallas_call(
        _edges_bn_body,
        grid=(_E // bm,),
        in_specs=[
            pl.BlockSpec((bm, _DIM), lambda i: (i, 0)),
            pl.BlockSpec((_NW, _DIM), lambda i: (0, 0)),
            pl.BlockSpec((_NW, _DIM), lambda i: (0, 0)),
            pl.BlockSpec((1, _DIM), lambda i: (0, 0)),
            pl.BlockSpec((1, _DIM), lambda i: (0, 0)),
        ],
        out_specs=pl.BlockSpec((bm, _DIM), lambda i: (i, 0)),
        out_shape=jax.ShapeDtypeStruct((_E, _DIM), _f32),
    )(m_arr, stats[:, 0, :], stats[:, 1, :], edge_gamma.reshape(1, _DIM),
      edge_beta.reshape(1, _DIM))

    return (out_nodes, out_edges)


# speedup vs baseline: 2.7437x; 1.0694x over previous
import functools

import jax
import jax.numpy as jnp
from jax import lax
from jax.experimental import pallas as pl
from jax.experimental.pallas import tpu as pltpu
from jax.experimental.pallas import tpu_sc as plsc

_f32 = jnp.float32

_N = 10000
_E = 320000
_DIM = 128

_NC = 2
_NS = 16
_NW = _NC * _NS

_CE = 32
_EPW = _E // _NW
_NPAIR = 156
_CREM = 16
_NPAD = 10112
_ROWS_PER_TILE = _NPAD // _NS
_INV_SQRT_D = 0.08838834764831845
_SL = _INV_SQRT_D * 1.4426950408889634
_MAGIC = 12582912.0
_MAGIC_I = 1262485504
_P2 = (1.00000008, 0.69312103, 0.24022107, 0.05592204, 0.00967604)


def _tables_body(x_ref, wt12_ref, dwt_ref, swt_ref, gb_ref, db_ref, sb_ref,
                 ta_ref, tbd_ref, sl_ref):
    x = x_ref[...]
    ta_ref[...] = jnp.dot(x, wt12_ref[:_DIM], preferred_element_type=_f32)
    tbd_ref[:, :_DIM] = (
        jnp.dot(x, wt12_ref[_DIM:], preferred_element_type=_f32) + gb_ref[...])
    tbd_ref[:, _DIM:] = (
        jnp.dot(x, dwt_ref[...], preferred_element_type=_f32) + db_ref[...])
    sl_ref[...] = jnp.dot(x, swt_ref[...], preferred_element_type=_f32) + sb_ref[...]


def _ec_body(ea_ref, wt3_ref, ec_ref):
    ec_ref[...] = jnp.dot(ea_ref[...], wt3_ref[...], preferred_element_type=_f32)


def _edge_body(src_hbm, dst_hbm, ta_hbm, tbd_hbm, ec_hbm,
               m_hbm, agg_hbm, stats_hbm,
               sa0_v, da0_v, sa1_v, da1_v, sb0_v, db0_v, sb1_v, db1_v,
               se_v, de_v,
               aa_v, ab_v, bda_v, bdb_v, eca_v, ecb_v,
               m_v, cta_v, ctb_v,
               agg_sh, sem_i, sem_ga, sem_gb, sem_m, sem_cta, sem_ctb):
    c = lax.axis_index("c")
    s = lax.axis_index("s")
    base = c * (_E // _NC) + s * _EPW
    row0 = s * _ROWS_PER_TILE

    def _zct(i, _):
        r = i // 8
        sl = pl.ds((i % 8) * 16, 16)
        cta_v[r, sl] = jnp.zeros((16,), _f32)
        ctb_v[r, sl] = jnp.zeros((16,), _f32)
        return 0
    lax.fori_loop(0, _CE * 8, _zct, 0)
    nz = _ROWS_PER_TILE // _CE
    rz = _ROWS_PER_TILE - nz * _CE
    for t in range(nz):
        pltpu.async_copy(cta_v, agg_sh.at[pl.ds(row0 + t * _CE, _CE)], sem_m)
    pltpu.async_copy(cta_v.at[pl.ds(0, rz)],
                     agg_sh.at[pl.ds(row0 + nz * _CE, rz)], sem_m)
    for t in range(nz):
        pltpu.make_async_copy(
            cta_v, agg_sh.at[pl.ds(row0 + t * _CE, _CE)], sem_m).wait()
    pltpu.make_async_copy(
        cta_v.at[pl.ds(0, rz)],
        agg_sh.at[pl.ds(row0 + nz * _CE, rz)], sem_m).wait()
    plsc.subcore_barrier()

    def _issue_idx(eb, si_v, di_v, nrow):
        c1 = pltpu.async_copy(src_hbm.at[pl.ds(eb, nrow)], si_v, sem_i)
        c2 = pltpu.async_copy(dst_hbm.at[pl.ds(eb, nrow)], di_v, sem_i)
        c1.wait()
        c2.wait()

    def _start_idx(eb, si_v, di_v):
        pltpu.async_copy(src_hbm.at[pl.ds(eb, _CE)], si_v, sem_i)
        pltpu.async_copy(dst_hbm.at[pl.ds(eb, _CE)], di_v, sem_i)

    def _finish_idx(eb, si_v, di_v):
        pltpu.make_async_copy(src_hbm.at[pl.ds(eb, _CE)], si_v, sem_i).wait()
        pltpu.make_async_copy(dst_hbm.at[pl.ds(eb, _CE)], di_v, sem_i).wait()

    def _issue_gathers(eb, si_v, di_v, a_v, bd_v, ec_v, sem):
        pltpu.async_copy(ta_hbm.at[si_v], a_v, sem)
        pltpu.async_copy(tbd_hbm.at[di_v], bd_v, sem)
        pltpu.async_copy(ec_hbm.at[pl.ds(eb, _CE)], ec_v, sem)

    def _wait_gathers(eb, si_v, di_v, a_v, bd_v, ec_v, sem):
        pltpu.make_async_copy(ta_hbm.at[si_v], a_v, sem).wait()
        pltpu.make_async_copy(tbd_hbm.at[di_v], bd_v, sem).wait()
        pltpu.make_async_copy(ec_hbm.at[pl.ds(eb, _CE)], ec_v, sem).wait()

    def _wait_m(nrow):
        pltpu.make_async_copy(
            m_v.at[pl.ds(0, nrow)], m_hbm.at[pl.ds(base, nrow)], sem_m).wait()

    def _wait_ct(ct_v, si_v, sem):
        pltpu.make_async_copy(ct_v, agg_sh.at[si_v], sem).wait()

    def _mk_row(a_v, bd_v, ec_v, ct_v):
        def _row(r, acc):
            accs = list(acc)
            for blk in range(2):
                js = range(4 * blk, 4 * blk + 4)
                sls = [pl.ds(j * 16, 16) for j in js]
                a = [a_v[r, sl] for sl in sls]
                b = [bd_v[r, sl] for sl in sls]
                e = [ec_v[r, sl] for sl in sls]
                m = [a[t] + b[t] for t in range(4)]
                m = [m[t] + e[t] for t in range(4)]
                for t, j in enumerate(js):
                    m_v[r, sls[t]] = m[t]
                am = [jnp.abs(m[t]) for t in range(4)]
                z = [am[t] * (-_SL) for t in range(4)]
                z = [jnp.maximum(z[t], -126.0) for t in range(4)]
                zm = [z[t] + _MAGIC for t in range(4)]
                nf = [zm[t] - _MAGIC for t in range(4)]
                fr = [z[t] - nf[t] for t in range(4)]
                ni = [lax.bitcast_convert_type(zm[t], jnp.int32) - _MAGIC_I
                      for t in range(4)]
                sh = [lax.shift_left(ni[t] + 127, 23) for t in range(4)]
                e2 = [lax.bitcast_convert_type(sh[t], _f32) for t in range(4)]
                p = [fr[t] * _P2[4] + _P2[3] for t in range(4)]
                p = [p[t] * fr[t] + _P2[2] for t in range(4)]
                p = [p[t] * fr[t] + _P2[1] for t in range(4)]
                p = [p[t] * fr[t] + _P2[0] for t in range(4)]
                y = [p[t] * e2[t] + 1.0 for t in range(4)]
                rc = [1.4117647 - 0.47058824 * y[t] for t in range(4)]
                w = [2.0 - y[t] * rc[t] for t in range(4)]
                rc = [rc[t] * w[t] for t in range(4)]
                w = [2.0 - y[t] * rc[t] for t in range(4)]
                rc = [rc[t] * w[t] for t in range(4)]
                ge = [m[t] >= 0.0 for t in range(4)]
                om = [1.0 - rc[t] for t in range(4)]
                sg = [jnp.where(ge[t], rc[t], om[t]) for t in range(4)]
                d = [bd_v[r, pl.ds(_DIM + j * 16, 16)] for j in js]
                ct = [sg[t] * d[t] for t in range(4)]
                for t, j in enumerate(js):
                    ct_v[r, sls[t]] = ct[t]
                for t, j in enumerate(js):
                    accs[2 * j] = accs[2 * j] + m[t]
                sq = [m[t] * m[t] for t in range(4)]
                for t, j in enumerate(js):
                    accs[2 * j + 1] = accs[2 * j + 1] + sq[t]
            return tuple(accs)
        return _row

    _row_a = _mk_row(aa_v, bda_v, eca_v, cta_v)
    _row_b = _mk_row(ab_v, bdb_v, ecb_v, ctb_v)

    _issue_idx(base, sa0_v, da0_v, _CE)
    _issue_gathers(base, sa0_v, da0_v, aa_v, bda_v, eca_v, sem_ga)
    _issue_idx(base + _CE, sb0_v, db0_v, _CE)
    _issue_gathers(base + _CE, sb0_v, db0_v, ab_v, bdb_v, ecb_v, sem_gb)
    pltpu.async_copy(cta_v, agg_sh.at[sa0_v], sem_cta, add=True)
    pltpu.async_copy(ctb_v, agg_sh.at[sb0_v], sem_ctb, add=True)
    pltpu.async_copy(m_v, agg_hbm.at[c, pl.ds(_N + 16, _CE)], sem_m)

    def _chunk(k, acc, row_fn, a_v, bd_v, ec_v, ct_v, si_v, di_v,
               pf, pf_si, pf_di):
        eb = base + k * _CE
        _wait_gathers(eb, si_v, di_v, a_v, bd_v, ec_v,
                      sem_ga if ct_v is cta_v else sem_gb)
        _wait_ct(ct_v, si_v, sem_cta if ct_v is cta_v else sem_ctb)
        if pf:
            _start_idx(eb + 2 * _CE, pf_si, pf_di)
        _wait_m(_CE)
        acc = plsc.parallel_loop(0, _CE, carry=acc)(row_fn)
        pltpu.async_copy(m_v, m_hbm.at[pl.ds(eb, _CE)], sem_m)
        pltpu.async_copy(ct_v, agg_sh.at[si_v],
                         sem_cta if ct_v is cta_v else sem_ctb, add=True)
        if pf:
            _finish_idx(eb + 2 * _CE, pf_si, pf_di)
            _issue_gathers(eb + 2 * _CE, pf_si, pf_di, a_v, bd_v, ec_v,
                           sem_ga if ct_v is cta_v else sem_gb)
        return acc

    def _quad(i, acc, pf_tail):
        k0 = 4 * i
        acc = _chunk(k0, acc, _row_a, aa_v, bda_v, eca_v, cta_v,
                     sa0_v, da0_v, True, sa1_v, da1_v)
        acc = _chunk(k0 + 1, acc, _row_b, ab_v, bdb_v, ecb_v, ctb_v,
                     sb0_v, db0_v, True, sb1_v, db1_v)
        acc = _chunk(k0 + 2, acc, _row_a, aa_v, bda_v, eca_v, cta_v,
                     sa1_v, da1_v, pf_tail, sa0_v, da0_v)
        acc = _chunk(k0 + 3, acc, _row_b, ab_v, bdb_v, ecb_v, ctb_v,
                     sb1_v, db1_v, pf_tail, sb0_v, db0_v)
        return acc

    zero16 = jnp.zeros((16,), _f32)
    nquad = _NPAIR // 2
    acc = lax.fori_loop(0, nquad - 1,
                        lambda i, a: _quad(i, a, True), (zero16,) * 16)
    acc = _quad(nquad - 1, acc, False)

    eb_r = base + 2 * _NPAIR * _CE
    _issue_idx(eb_r, se_v, de_v, _CREM)
    pltpu.async_copy(ta_hbm.at[se_v], aa_v.at[pl.ds(0, _CREM)], sem_ga)
    pltpu.async_copy(tbd_hbm.at[de_v], bda_v.at[pl.ds(0, _CREM)], sem_ga)
    pltpu.async_copy(ec_hbm.at[pl.ds(eb_r, _CREM)], eca_v.at[pl.ds(0, _CREM)],
                     sem_ga)
    _wait_ct(cta_v, sa1_v, sem_cta)
    _wait_m(_CE)
    pltpu.make_async_copy(ta_hbm.at[se_v], aa_v.at[pl.ds(0, _CREM)],
                          sem_ga).wait()
    pltpu.make_async_copy(tbd_hbm.at[de_v], bda_v.at[pl.ds(0, _CREM)],
                          sem_ga).wait()
    pltpu.make_async_copy(ec_hbm.at[pl.ds(eb_r, _CREM)],
                          eca_v.at[pl.ds(0, _CREM)], sem_ga).wait()
    acc = plsc.parallel_loop(0, _CREM, carry=acc)(_row_a)
    pltpu.async_copy(m_v.at[pl.ds(0, _CREM)], m_hbm.at[pl.ds(eb_r, _CREM)],
                     sem_m)
    pltpu.async_copy(cta_v.at[pl.ds(0, _CREM)], agg_sh.at[se_v], sem_cta,
                     add=True)
    _wait_m(_CREM)
    _wait_ct(ctb_v, sb1_v, sem_ctb)
    pltpu.make_async_copy(cta_v.at[pl.ds(0, _CREM)], agg_sh.at[se_v],
                          sem_cta).wait()

    def _zst(i, _):
        cta_v[i // 8, pl.ds((i % 8) * 16, 16)] = jnp.zeros((16,), _f32)
        return 0
    lax.fori_loop(0, _CREM * 8, _zst, 0)
    for j in range(8):
        cta_v[0, pl.ds(j * 16, 16)] = acc[2 * j]
        cta_v[1, pl.ds(j * 16, 16)] = acc[2 * j + 1]
    wid = s * _NC + c
    pltpu.sync_copy(cta_v.at[pl.ds(0, _CREM)], stats_hbm.at[wid])

    plsc.subcore_barrier()
    pltpu.sync_copy(agg_sh.at[pl.ds(row0, _ROWS_PER_TILE)],
                    agg_hbm.at[c, pl.ds(row0, _ROWS_PER_TILE)])


def _nodes_body(sl_ref, a0_ref, a1_ref, g_ref, b_ref, o_ref):
    h = sl_ref[...] + a0_ref[...] + a1_ref[...]
    mean = jnp.mean(h, axis=0, keepdims=True)
    var = jnp.mean((h - mean) * (h - mean), axis=0, keepdims=True)
    o_ref[...] = jnp.maximum(
        (h - mean) * lax.rsqrt(var + 1e-5) * g_ref[...] + b_ref[...], 0.0)


def _edges_bn_body(m_ref, ssum_ref, ssq_ref, g_ref, b_ref, o_ref):
    mean = jnp.sum(ssum_ref[...], axis=0, keepdims=True) * (1.0 / _E)
    msq = jnp.sum(ssq_ref[...], axis=0, keepdims=True) * (1.0 / _E)
    var = msq - mean * mean
    rstd = lax.rsqrt(var + 1e-5)
    o_ref[...] = jnp.maximum(
        (m_ref[...] - mean) * rstd * g_ref[...] + b_ref[...], 0.0)


def kernel(x, edge_index, edge_attr, gate_W, gate_b, src_W, src_b, dst_W,
           dst_b, node_gamma, node_beta, edge_gamma, edge_beta):
    src_idx = edge_index[0]
    dst_idx = edge_index[1]
    wt = gate_W.T
    wt12 = wt[:2 * _DIM]
    wt3 = wt[2 * _DIM:]
    gb2 = gate_b.reshape(1, _DIM)
    db2 = dst_b.reshape(1, _DIM)
    sb2 = src_b.reshape(1, _DIM)

    ta, tbd, src_lin = pl.pallas_call(
        _tables_body,
        out_shape=[
            jax.ShapeDtypeStruct((_N, _DIM), _f32),
            jax.ShapeDtypeStruct((_N, 2 * _DIM), _f32),
            jax.ShapeDtypeStruct((_N, _DIM), _f32),
        ],
    )(x, wt12, dst_W.T, src_W.T, gb2, db2, sb2)

    be = 4000
    ec = pl.pallas_call(
        _ec_body,
        grid=(_E // be,),
        in_specs=[
            pl.BlockSpec((be, _DIM), lambda i: (i, 0)),
            pl.BlockSpec((_DIM, _DIM), lambda i: (0, 0)),
        ],
        out_specs=pl.BlockSpec((be, _DIM), lambda i: (i, 0)),
        out_shape=jax.ShapeDtypeStruct((_E, _DIM), _f32),
    )(edge_attr, wt3)

    mesh = plsc.VectorSubcoreMesh(core_axis_name="c", subcore_axis_name="s")
    m_arr, agg2, stats = pl.kernel(
        _edge_body,
        out_type=[
            jax.ShapeDtypeStruct((_E, _DIM), _f32),
            jax.ShapeDtypeStruct((_NC, _NPAD, _DIM), _f32),
            jax.ShapeDtypeStruct((_NW, _CREM, _DIM), _f32),
        ],
        mesh=mesh,
        scratch_types=(
            [pltpu.VMEM((_CE,), jnp.int32)] * 8
            + [pltpu.VMEM((_CREM,), jnp.int32)] * 2
            + [
                pltpu.VMEM((_CE, _DIM), _f32),
                pltpu.VMEM((_CE, _DIM), _f32),
                pltpu.VMEM((_CE, 2 * _DIM), _f32),
                pltpu.VMEM((_CE, 2 * _DIM), _f32),
                pltpu.VMEM((_CE, _DIM), _f32),
                pltpu.VMEM((_CE, _DIM), _f32),
                pltpu.VMEM((_CE, _DIM), _f32),
                pltpu.VMEM((_CE, _DIM), _f32),
                pltpu.VMEM((_CE, _DIM), _f32),
                pltpu.VMEM_SHARED((_NPAD, _DIM), _f32),
            ]
            + [pltpu.SemaphoreType.DMA] * 6
        ),
    )(src_idx, dst_idx, ta, tbd, ec)

    out_nodes = pl.pallas_call(
        _nodes_body,
        out_shape=jax.ShapeDtypeStruct((_N, _DIM), _f32),
    )(src_lin, agg2[0, :_N], agg2[1, :_N], node_gamma.reshape(1, _DIM),
      node_beta.reshape(1, _DIM))

    bm = 4000
    out_edges = pl.pallas_call(
        _edges_bn_body,
        grid=(_E // bm,),
        in_specs=[
            pl.BlockSpec((bm, _DIM), lambda i: (i, 0)),
            pl.BlockSpec((_NW, _DIM), lambda i: (0, 0)),
            pl.BlockSpec((_NW, _DIM), lambda i: (0, 0)),
            pl.BlockSpec((1, _DIM), lambda i: (0, 0)),
            pl.BlockSpec((1, _DIM), lambda i: (0, 0)),
        ],
        out_specs=pl.BlockSpec((bm, _DIM), lambda i: (i, 0)),
        out_shape=jax.ShapeDtypeStruct((_E, _DIM), _f32),
    )(m_arr, stats[:, 0, :], stats[:, 1, :], edge_gamma.reshape(1, _DIM),
      edge_beta.reshape(1, _DIM))

    return (out_nodes, out_edges)


# speedup vs baseline: 2.8798x; 1.0496x over previous
import functools

import jax
import jax.numpy as jnp
from jax import lax
from jax.experimental import pallas as pl
from jax.experimental.pallas import tpu as pltpu
from jax.experimental.pallas import tpu_sc as plsc

_f32 = jnp.float32

_N = 10000
_E = 320000
_DIM = 128

_NC = 2
_NS = 16
_NW = _NC * _NS

_CE = 32
_EPW = _E // _NW
_NPAIR = 156
_CREM = 16
_NPAD = 10112
_ROWS_PER_TILE = _NPAD // _NS
_INV_SQRT_D = 0.08838834764831845
_SL = _INV_SQRT_D * 1.4426950408889634
_MAGIC = 12582912.0
_MAGIC_I = 1262485504
_P2 = (1.00000008, 0.69312103, 0.24022107, 0.05592204, 0.00967604)


def _tables_body(x_ref, wt12_ref, dwt_ref, swt_ref, gb_ref, db_ref, sb_ref,
                 ta_ref, tbd_ref, sl_ref):
    x = x_ref[...]
    ta_ref[...] = jnp.dot(x, wt12_ref[:_DIM], preferred_element_type=_f32)
    tbd_ref[:, :_DIM] = (
        jnp.dot(x, wt12_ref[_DIM:], preferred_element_type=_f32) + gb_ref[...])
    tbd_ref[:, _DIM:] = (
        jnp.dot(x, dwt_ref[...], preferred_element_type=_f32) + db_ref[...])
    sl_ref[...] = jnp.dot(x, swt_ref[...], preferred_element_type=_f32) + sb_ref[...]


def _ec_body(ea_ref, wt3_ref, ec_ref):
    ec_ref[...] = jnp.dot(ea_ref[...], wt3_ref[...], preferred_element_type=_f32)


def _edge_body(src_hbm, dst_hbm, ta_hbm, tbd_hbm, ec_hbm,
               m_hbm, agg_hbm, stats_hbm,
               sa0_v, da0_v, sa1_v, da1_v, sb0_v, db0_v, sb1_v, db1_v,
               se_v, de_v,
               aa_v, ab_v, bda_v, bdb_v, eca_v, ecb_v,
               m_v, cta_v, ctb_v,
               agg_sh, sem_i, sem_ga, sem_gb, sem_m, sem_cta, sem_ctb):
    c = lax.axis_index("c")
    s = lax.axis_index("s")
    base = c * (_E // _NC) + s * _EPW
    row0 = s * _ROWS_PER_TILE

    def _zct(i, _):
        r = i // 8
        sl = pl.ds((i % 8) * 16, 16)
        cta_v[r, sl] = jnp.zeros((16,), _f32)
        ctb_v[r, sl] = jnp.zeros((16,), _f32)
        return 0
    lax.fori_loop(0, _CE * 8, _zct, 0)
    nz = _ROWS_PER_TILE // _CE
    rz = _ROWS_PER_TILE - nz * _CE
    for t in range(nz):
        pltpu.async_copy(cta_v, agg_sh.at[pl.ds(row0 + t * _CE, _CE)], sem_m)
    pltpu.async_copy(cta_v.at[pl.ds(0, rz)],
                     agg_sh.at[pl.ds(row0 + nz * _CE, rz)], sem_m)
    for t in range(nz):
        pltpu.make_async_copy(
            cta_v, agg_sh.at[pl.ds(row0 + t * _CE, _CE)], sem_m).wait()
    pltpu.make_async_copy(
        cta_v.at[pl.ds(0, rz)],
        agg_sh.at[pl.ds(row0 + nz * _CE, rz)], sem_m).wait()
    plsc.subcore_barrier()

    def _issue_idx(eb, si_v, di_v, nrow):
        c1 = pltpu.async_copy(src_hbm.at[pl.ds(eb, nrow)], si_v, sem_i)
        c2 = pltpu.async_copy(dst_hbm.at[pl.ds(eb, nrow)], di_v, sem_i)
        c1.wait()
        c2.wait()

    def _issue_gathers(eb, si_v, di_v, a_v, bd_v, ec_v, sem):
        pltpu.async_copy(ta_hbm.at[si_v], a_v, sem)
        pltpu.async_copy(tbd_hbm.at[di_v], bd_v, sem)
        pltpu.async_copy(ec_hbm.at[pl.ds(eb, _CE)], ec_v, sem)

    def _wait_gathers(eb, si_v, di_v, a_v, bd_v, ec_v, sem):
        pltpu.make_async_copy(ta_hbm.at[si_v], a_v, sem).wait()
        pltpu.make_async_copy(tbd_hbm.at[di_v], bd_v, sem).wait()
        pltpu.make_async_copy(ec_hbm.at[pl.ds(eb, _CE)], ec_v, sem).wait()

    def _wait_m(nrow):
        pltpu.make_async_copy(
            m_v.at[pl.ds(0, nrow)], m_hbm.at[pl.ds(base, nrow)], sem_m).wait()

    def _wait_ct(ct_v, si_v, sem):
        pltpu.make_async_copy(ct_v, agg_sh.at[si_v], sem).wait()

    def _mk_row(a_v, bd_v, ec_v, ct_v):
        def _row(r, acc):
            accs = list(acc)
            for blk in range(2):
                js = range(4 * blk, 4 * blk + 4)
                sls = [pl.ds(j * 16, 16) for j in js]
                a = [a_v[r, sl] for sl in sls]
                b = [bd_v[r, sl] for sl in sls]
                e = [ec_v[r, sl] for sl in sls]
                m = [a[t] + b[t] for t in range(4)]
                m = [m[t] + e[t] for t in range(4)]
                for t, j in enumerate(js):
                    m_v[r, sls[t]] = m[t]
                am = [jnp.abs(m[t]) for t in range(4)]
                z = [am[t] * (-_INV_SQRT_D) for t in range(4)]
                ex = [jnp.exp(z[t]) for t in range(4)]
                y = [ex[t] + 1.0 for t in range(4)]
                rc = [1.4117647 - 0.47058824 * y[t] for t in range(4)]
                w = [2.0 - y[t] * rc[t] for t in range(4)]
                rc = [rc[t] * w[t] for t in range(4)]
                w = [2.0 - y[t] * rc[t] for t in range(4)]
                rc = [rc[t] * w[t] for t in range(4)]
                ge = [m[t] >= 0.0 for t in range(4)]
                om = [1.0 - rc[t] for t in range(4)]
                sg = [jnp.where(ge[t], rc[t], om[t]) for t in range(4)]
                d = [bd_v[r, pl.ds(_DIM + j * 16, 16)] for j in js]
                ct = [sg[t] * d[t] for t in range(4)]
                for t, j in enumerate(js):
                    ct_v[r, sls[t]] = ct[t]
                for t, j in enumerate(js):
                    accs[2 * j] = accs[2 * j] + m[t]
                sq = [m[t] * m[t] for t in range(4)]
                for t, j in enumerate(js):
                    accs[2 * j + 1] = accs[2 * j + 1] + sq[t]
            return tuple(accs)
        return _row

    _row_a = _mk_row(aa_v, bda_v, eca_v, cta_v)
    _row_b = _mk_row(ab_v, bdb_v, ecb_v, ctb_v)

    _issue_idx(base, sa0_v, da0_v, _CE)
    _issue_gathers(base, sa0_v, da0_v, aa_v, bda_v, eca_v, sem_ga)
    _issue_idx(base + _CE, sb0_v, db0_v, _CE)
    _issue_gathers(base + _CE, sb0_v, db0_v, ab_v, bdb_v, ecb_v, sem_gb)
    pltpu.async_copy(cta_v, agg_sh.at[sa0_v], sem_cta, add=True)
    pltpu.async_copy(ctb_v, agg_sh.at[sb0_v], sem_ctb, add=True)
    pltpu.async_copy(m_v, agg_hbm.at[c, pl.ds(_N + 16, _CE)], sem_m)

    def _chunk(k, acc, row_fn, a_v, bd_v, ec_v, ct_v, si_v, di_v,
               pf, pf_si, pf_di):
        eb = base + k * _CE
        _wait_gathers(eb, si_v, di_v, a_v, bd_v, ec_v,
                      sem_ga if ct_v is cta_v else sem_gb)
        _wait_ct(ct_v, si_v, sem_cta if ct_v is cta_v else sem_ctb)
        _wait_m(_CE)
        acc = plsc.parallel_loop(0, _CE, carry=acc)(row_fn)
        pltpu.async_copy(m_v, m_hbm.at[pl.ds(eb, _CE)], sem_m)
        pltpu.async_copy(ct_v, agg_sh.at[si_v],
                         sem_cta if ct_v is cta_v else sem_ctb, add=True)
        if pf:
            _issue_idx(eb + 2 * _CE, pf_si, pf_di, _CE)
            _issue_gathers(eb + 2 * _CE, pf_si, pf_di, a_v, bd_v, ec_v,
                           sem_ga if ct_v is cta_v else sem_gb)
        return acc

    def _quad(i, acc, pf_tail):
        k0 = 4 * i
        acc = _chunk(k0, acc, _row_a, aa_v, bda_v, eca_v, cta_v,
                     sa0_v, da0_v, True, sa1_v, da1_v)
        acc = _chunk(k0 + 1, acc, _row_b, ab_v, bdb_v, ecb_v, ctb_v,
                     sb0_v, db0_v, True, sb1_v, db1_v)
        acc = _chunk(k0 + 2, acc, _row_a, aa_v, bda_v, eca_v, cta_v,
                     sa1_v, da1_v, pf_tail, sa0_v, da0_v)
        acc = _chunk(k0 + 3, acc, _row_b, ab_v, bdb_v, ecb_v, ctb_v,
                     sb1_v, db1_v, pf_tail, sb0_v, db0_v)
        return acc

    zero16 = jnp.zeros((16,), _f32)
    nquad = _NPAIR // 2
    acc = lax.fori_loop(0, nquad - 1,
                        lambda i, a: _quad(i, a, True), (zero16,) * 16)
    acc = _quad(nquad - 1, acc, False)

    eb_r = base + 2 * _NPAIR * _CE
    _issue_idx(eb_r, se_v, de_v, _CREM)
    pltpu.async_copy(ta_hbm.at[se_v], aa_v.at[pl.ds(0, _CREM)], sem_ga)
    pltpu.async_copy(tbd_hbm.at[de_v], bda_v.at[pl.ds(0, _CREM)], sem_ga)
    pltpu.async_copy(ec_hbm.at[pl.ds(eb_r, _CREM)], eca_v.at[pl.ds(0, _CREM)],
                     sem_ga)
    _wait_ct(cta_v, sa1_v, sem_cta)
    _wait_m(_CE)
    pltpu.make_async_copy(ta_hbm.at[se_v], aa_v.at[pl.ds(0, _CREM)],
                          sem_ga).wait()
    pltpu.make_async_copy(tbd_hbm.at[de_v], bda_v.at[pl.ds(0, _CREM)],
                          sem_ga).wait()
    pltpu.make_async_copy(ec_hbm.at[pl.ds(eb_r, _CREM)],
                          eca_v.at[pl.ds(0, _CREM)], sem_ga).wait()
    acc = plsc.parallel_loop(0, _CREM, carry=acc)(_row_a)
    pltpu.async_copy(m_v.at[pl.ds(0, _CREM)], m_hbm.at[pl.ds(eb_r, _CREM)],
                     sem_m)
    pltpu.async_copy(cta_v.at[pl.ds(0, _CREM)], agg_sh.at[se_v], sem_cta,
                     add=True)
    _wait_m(_CREM)
    _wait_ct(ctb_v, sb1_v, sem_ctb)
    pltpu.make_async_copy(cta_v.at[pl.ds(0, _CREM)], agg_sh.at[se_v],
                          sem_cta).wait()

    def _zst(i, _):
        cta_v[i // 8, pl.ds((i % 8) * 16, 16)] = jnp.zeros((16,), _f32)
        return 0
    lax.fori_loop(0, _CREM * 8, _zst, 0)
    for j in range(8):
        cta_v[0, pl.ds(j * 16, 16)] = acc[2 * j]
        cta_v[1, pl.ds(j * 16, 16)] = acc[2 * j + 1]
    wid = s * _NC + c
    pltpu.sync_copy(cta_v.at[pl.ds(0, _CREM)], stats_hbm.at[wid])

    plsc.subcore_barrier()
    pltpu.sync_copy(agg_sh.at[pl.ds(row0, _ROWS_PER_TILE)],
                    agg_hbm.at[c, pl.ds(row0, _ROWS_PER_TILE)])


def _nodes_body(sl_ref, a0_ref, a1_ref, g_ref, b_ref, o_ref):
    h = sl_ref[...] + a0_ref[...] + a1_ref[...]
    mean = jnp.mean(h, axis=0, keepdims=True)
    var = jnp.mean((h - mean) * (h - mean), axis=0, keepdims=True)
    o_ref[...] = jnp.maximum(
        (h - mean) * lax.rsqrt(var + 1e-5) * g_ref[...] + b_ref[...], 0.0)


def _edges_bn_body(m_ref, ssum_ref, ssq_ref, g_ref, b_ref, o_ref):
    mean = jnp.sum(ssum_ref[...], axis=0, keepdims=True) * (1.0 / _E)
    msq = jnp.sum(ssq_ref[...], axis=0, keepdims=True) * (1.0 / _E)
    var = msq - mean * mean
    rstd = lax.rsqrt(var + 1e-5)
    o_ref[...] = jnp.maximum(
        (m_ref[...] - mean) * rstd * g_ref[...] + b_ref[...], 0.0)


def kernel(x, edge_index, edge_attr, gate_W, gate_b, src_W, src_b, dst_W,
           dst_b, node_gamma, node_beta, edge_gamma, edge_beta):
    src_idx = edge_index[0]
    dst_idx = edge_index[1]
    wt = gate_W.T
    wt12 = wt[:2 * _DIM]
    wt3 = wt[2 * _DIM:]
    gb2 = gate_b.reshape(1, _DIM)
    db2 = dst_b.reshape(1, _DIM)
    sb2 = src_b.reshape(1, _DIM)

    ta, tbd, src_lin = pl.pallas_call(
        _tables_body,
        out_shape=[
            jax.ShapeDtypeStruct((_N, _DIM), _f32),
            jax.ShapeDtypeStruct((_N, 2 * _DIM), _f32),
            jax.ShapeDtypeStruct((_N, _DIM), _f32),
        ],
    )(x, wt12, dst_W.T, src_W.T, gb2, db2, sb2)

    be = 4000
    ec = pl.pallas_call(
        _ec_body,
        grid=(_E // be,),
        in_specs=[
            pl.BlockSpec((be, _DIM), lambda i: (i, 0)),
            pl.BlockSpec((_DIM, _DIM), lambda i: (0, 0)),
        ],
        out_specs=pl.BlockSpec((be, _DIM), lambda i: (i, 0)),
        out_shape=jax.ShapeDtypeStruct((_E, _DIM), _f32),
    )(edge_attr, wt3)

    mesh = plsc.VectorSubcoreMesh(core_axis_name="c", subcore_axis_name="s")
    m_arr, agg2, stats = pl.kernel(
        _edge_body,
        out_type=[
            jax.ShapeDtypeStruct((_E, _DIM), _f32),
            jax.ShapeDtypeStruct((_NC, _NPAD, _DIM), _f32),
            jax.ShapeDtypeStruct((_NW, _CREM, _DIM), _f32),
        ],
        mesh=mesh,
        scratch_types=(
            [pltpu.VMEM((_CE,), jnp.int32)] * 8
            + [pltpu.VMEM((_CREM,), jnp.int32)] * 2
            + [
                pltpu.VMEM((_CE, _DIM), _f32),
                pltpu.VMEM((_CE, _DIM), _f32),
                pltpu.VMEM((_CE, 2 * _DIM), _f32),
                pltpu.VMEM((_CE, 2 * _DIM), _f32),
                pltpu.VMEM((_CE, _DIM), _f32),
                pltpu.VMEM((_CE, _DIM), _f32),
                pltpu.VMEM((_CE, _DIM), _f32),
                pltpu.VMEM((_CE, _DIM), _f32),
                pltpu.VMEM((_CE, _DIM), _f32),
                pltpu.VMEM_SHARED((_NPAD, _DIM), _f32),
            ]
            + [pltpu.SemaphoreType.DMA] * 6
        ),
    )(src_idx, dst_idx, ta, tbd, ec)

    out_nodes = pl.pallas_call(
        _nodes_body,
        out_shape=jax.ShapeDtypeStruct((_N, _DIM), _f32),
    )(src_lin, agg2[0, :_N], agg2[1, :_N], node_gamma.reshape(1, _DIM),
      node_beta.reshape(1, _DIM))

    bm = 4000
    out_edges = pl.pallas_call(
        _edges_bn_body,
        grid=(_E // bm,),
        in_specs=[
            pl.BlockSpec((bm, _DIM), lambda i: (i, 0)),
            pl.BlockSpec((_NW, _DIM), lambda i: (0, 0)),
            pl.BlockSpec((_NW, _DIM), lambda i: (0, 0)),
            pl.BlockSpec((1, _DIM), lambda i: (0, 0)),
            pl.BlockSpec((1, _DIM), lambda i: (0, 0)),
        ],
        out_specs=pl.BlockSpec((bm, _DIM), lambda i: (i, 0)),
        out_shape=jax.ShapeDtypeStruct((_E, _DIM), _f32),
    )(m_arr, stats[:, 0, :], stats[:, 1, :], edge_gamma.reshape(1, _DIM),
      edge_beta.reshape(1, _DIM))

    return (out_nodes, out_edges)
